# baseline (speedup 1.0000x reference)
import jax
import jax.numpy as jnp
from jax.experimental import pallas as pl

DEPTH = 1000
ROWS_PER_BLOCK = 256


def _onehot_block(idx_ref, out_ref):
    idx = idx_ref[...]
    iota = jax.lax.broadcasted_iota(idx.dtype, out_ref.shape, 2)
    out_ref[...] = (idx[:, :, None] == iota).astype(jnp.float32)


def kernel(inputs):
    n, m = inputs.shape
    r = ROWS_PER_BLOCK
    grid = (n // r,)
    return pl.pallas_call(
        _onehot_block,
        grid=grid,
        in_specs=[pl.BlockSpec((r, m), lambda i: (i, 0))],
        out_specs=pl.BlockSpec((r, m, DEPTH), lambda i: (i, 0, 0)),
        out_shape=jax.ShapeDtypeStruct((n, m, DEPTH), jnp.float32),
    )(inputs)
